# Initial kernel scaffold; baseline (speedup 1.0000x reference)
import jax
import jax.numpy as jnp
from jax.experimental import pallas as pl


def kernel(node_feat, edge_index, W1, b1, s1, W2, b2, s2, cls, bias_dec):
    raise NotImplementedError("write your pallas kernel here")


# speedup vs baseline: 2.6092x; 2.6092x over previous
import functools

import jax
import jax.numpy as jnp
from jax import lax
from jax.experimental import pallas as pl
from jax.experimental.pallas import tpu as pltpu
from jax.experimental.pallas import tpu_sc as plsc

N = 10000
NP = 10240
E = 320000
D = 128
DH = D + 1
NCLS = 7
CCURV = 1.0

NC = 2
NS = 16
NW = NC * NS
CROWS = 5
DHP = NW * CROWS
EC = 10000
NB = 2048

_prec = jax.lax.Precision.HIGHEST


def _lorentz_normalize(h, sval):
    h0 = h[0:1, :]
    sig = 1.0 / (1.0 + jnp.exp(-h0))
    time = sig * sval + (jnp.sqrt(CCURV) + 0.5)
    sqall = jnp.sum(h * h, axis=0, keepdims=True)
    sq = jnp.maximum(sqall - h0 * h0, 1e-8)
    scale = (time * time - CCURV) / sq
    fac = jnp.sqrt(jnp.maximum(scale, 1e-8))
    rows = lax.broadcasted_iota(jnp.int32, h.shape, 0)
    return jnp.where(rows == 0, time, h * fac)


def _agg_normalize(s):
    s0 = s[0:1, :]
    sqall = jnp.sum(s * s, axis=0, keepdims=True)
    inner = -(s0 * s0) + (sqall - s0 * s0)
    denom = jnp.sqrt(jnp.maximum(jnp.abs(-inner), 1e-8)) / jnp.sqrt(CCURV)
    return s / denom


def _tc_a_body(nf_ref, w1sp_ref, w1t_ref, b1_ref, sv_ref, out_ref):
    nf = nf_ref[...]
    sq = jnp.sum(nf * nf, axis=1, keepdims=True)
    nrm = jnp.maximum(jnp.sqrt(sq), 1e-8)
    et = jnp.exp(nrm)
    emt = 1.0 / et
    time = 0.5 * (et + emt)
    space = nf * (0.5 * (et - emt) / nrm)
    h = lax.dot_general(w1sp_ref[...], space, (((1,), (1,)), ((), ())),
                        preferred_element_type=jnp.float32, precision=_prec)
    h = h + lax.dot_general(w1t_ref[...], time, (((1,), (1,)), ((), ())),
                            preferred_element_type=jnp.float32, precision=_prec)
    h = h + b1_ref[...]
    out_ref[...] = _lorentz_normalize(h, sv_ref[0, 0])


def _tc_b_body(sup_ref, w2_ref, b2_ref, sv_ref, out_ref):
    hagg = _agg_normalize(sup_ref[...])
    hr = jnp.maximum(hagg, 0.0)
    h = lax.dot_general(w2_ref[...], hr, (((1,), (0,)), ((), ())),
                        preferred_element_type=jnp.float32, precision=_prec)
    h = h + b2_ref[...]
    out_ref[...] = _lorentz_normalize(h, sv_ref[0, 0])


def _tc_c_body(sup_ref, cls_ref, bias_ref, out_ref):
    h = _agg_normalize(sup_ref[...])
    logits = lax.dot_general(h, cls_ref[...], (((0,), (1,)), ((), ())),
                             preferred_element_type=jnp.float32,
                             precision=_prec)
    out_ref[...] = 2.0 * CCURV + 2.0 * logits + bias_ref[...]


def _tc_stage_a(nf, w1sp, w1t, b1p, sv1):
    return pl.pallas_call(
        _tc_a_body,
        grid=(NP // NB,),
        in_specs=[
            pl.BlockSpec((NB, D), lambda i: (i, 0)),
            pl.BlockSpec((DHP, D), lambda i: (0, 0)),
            pl.BlockSpec((DHP, 1), lambda i: (0, 0)),
            pl.BlockSpec((DHP, 1), lambda i: (0, 0)),
            pl.BlockSpec((1, 1), lambda i: (0, 0)),
        ],
        out_specs=pl.BlockSpec((DHP, NB), lambda i: (0, i)),
        out_shape=jax.ShapeDtypeStruct((DHP, NP), jnp.float32),
    )(nf, w1sp, w1t, b1p, sv1)


def _tc_stage_b(sup, w2p, b2p, sv2):
    return pl.pallas_call(
        _tc_b_body,
        grid=(NP // NB,),
        in_specs=[
            pl.BlockSpec((DHP, NB), lambda i: (0, i)),
            pl.BlockSpec((DHP, DHP), lambda i: (0, 0)),
            pl.BlockSpec((DHP, 1), lambda i: (0, 0)),
            pl.BlockSpec((1, 1), lambda i: (0, 0)),
        ],
        out_specs=pl.BlockSpec((DHP, NB), lambda i: (0, i)),
        out_shape=jax.ShapeDtypeStruct((DHP, NP), jnp.float32),
    )(sup, w2p, b2p, sv2)


def _tc_stage_c(sup, clsm, biasp):
    return pl.pallas_call(
        _tc_c_body,
        grid=(NP // NB,),
        in_specs=[
            pl.BlockSpec((DHP, NB), lambda i: (0, i)),
            pl.BlockSpec((NCLS, DHP), lambda i: (0, 0)),
            pl.BlockSpec((1, NCLS), lambda i: (0, 0)),
        ],
        out_specs=pl.BlockSpec((NB, NCLS), lambda i: (i, 0)),
        out_shape=jax.ShapeDtypeStruct((NP, NCLS), jnp.float32),
    )(sup, clsm, biasp)


def _sc_seg_sum(hT, src, dst):
    mesh = plsc.VectorSubcoreMesh(core_axis_name="c", subcore_axis_name="s")

    @functools.partial(
        pl.kernel,
        out_type=jax.ShapeDtypeStruct((DHP, NP), jnp.float32),
        mesh=mesh,
        compiler_params=pltpu.CompilerParams(needs_layout_passes=False),
        scratch_types=[
            pltpu.VMEM((CROWS * NP,), jnp.float32),
            pltpu.VMEM((CROWS * NP,), jnp.float32),
            pltpu.VMEM((EC,), jnp.int32),
            pltpu.VMEM((EC,), jnp.int32),
        ],
    )
    def seg(hT_hbm, src_hbm, dst_hbm, out_hbm, tab, acc, sbuf, dbuf):
        wid = lax.axis_index("s") * NC + lax.axis_index("c")
        r0 = wid * CROWS

        for c in range(CROWS):
            pltpu.sync_copy(hT_hbm.at[r0 + c], tab.at[pl.ds(c * NP, NP)])

        zeros16 = jnp.zeros((16,), jnp.float32)

        def zero_body(j, _):
            acc[pl.ds(j * 16, 16)] = zeros16
            return 0

        lax.fori_loop(0, CROWS * NP // 16, zero_body, 0)

        def chunk_body(k, _):
            off = k * EC
            pltpu.sync_copy(src_hbm.at[pl.ds(off, EC)], sbuf)
            pltpu.sync_copy(dst_hbm.at[pl.ds(off, EC)], dbuf)

            def edge_body(j, _):
                s16 = sbuf[pl.ds(j * 16, 16)]
                d16 = dbuf[pl.ds(j * 16, 16)]
                for c in range(CROWS):
                    v = plsc.load_gather(tab, [s16 + (c * NP)])
                    plsc.addupdate_scatter(acc, [d16 + (c * NP)], v)
                return 0

            lax.fori_loop(0, EC // 16, edge_body, 0)
            return 0

        lax.fori_loop(0, E // EC, chunk_body, 0)

        for c in range(CROWS):
            pltpu.sync_copy(acc.at[pl.ds(c * NP, NP)], out_hbm.at[r0 + c])

    return seg(hT, src, dst)


def kernel(node_feat, edge_index, W1, b1, s1, W2, b2, s2, cls, bias_dec):
    f32 = jnp.float32
    src = edge_index[0]
    dst = edge_index[1]

    w1p = jnp.zeros((DHP, DH), f32).at[:DH].set(W1)
    w1sp = w1p[:, 1:]
    w1t = w1p[:, 0:1]
    b1p = jnp.zeros((DHP, 1), f32).at[:DH, 0].set(b1)
    sv1 = jnp.minimum(jnp.exp(s1), 10.0).reshape(1, 1).astype(f32)
    w2p = jnp.zeros((DHP, DHP), f32).at[:DH, :DH].set(W2)
    b2p = jnp.zeros((DHP, 1), f32).at[:DH, 0].set(b2)
    sv2 = jnp.minimum(jnp.exp(s2), 10.0).reshape(1, 1).astype(f32)
    clsm = jnp.zeros((NCLS, DHP), f32).at[:, :DH].set(cls)
    clsm = clsm.at[:, 0].mul(-1.0)
    biasp = bias_dec.reshape(1, NCLS).astype(f32)

    nfp = jnp.zeros((NP, D), f32).at[:N].set(node_feat)

    h1 = _tc_stage_a(nfp, w1sp, w1t, b1p, sv1)
    sup1 = _sc_seg_sum(h1, src, dst)
    h2 = _tc_stage_b(sup1, w2p, b2p, sv2)
    sup2 = _sc_seg_sum(h2, src, dst)
    return _tc_stage_c(sup2, clsm, biasp)[:N]


# speedup vs baseline: 2.8938x; 1.1091x over previous
import functools

import jax
import jax.numpy as jnp
from jax import lax
from jax.experimental import pallas as pl
from jax.experimental.pallas import tpu as pltpu
from jax.experimental.pallas import tpu_sc as plsc

N = 10000
NP = 10240
E = 320000
D = 128
DH = D + 1
NCLS = 7
CCURV = 1.0

NC = 2
NS = 16
NW = NC * NS
CROWS = 5
DHP = NW * CROWS
ECH = 4000
NCH = E // ECH
UNROLL = 5
NB = 2048

_prec = jax.lax.Precision.HIGHEST


def _lorentz_normalize(h, sval):
    h0 = h[0:1, :]
    sig = 1.0 / (1.0 + jnp.exp(-h0))
    time = sig * sval + (jnp.sqrt(CCURV) + 0.5)
    sqall = jnp.sum(h * h, axis=0, keepdims=True)
    sq = jnp.maximum(sqall - h0 * h0, 1e-8)
    scale = (time * time - CCURV) / sq
    fac = jnp.sqrt(jnp.maximum(scale, 1e-8))
    rows = lax.broadcasted_iota(jnp.int32, h.shape, 0)
    return jnp.where(rows == 0, time, h * fac)


def _agg_normalize(s):
    s0 = s[0:1, :]
    sqall = jnp.sum(s * s, axis=0, keepdims=True)
    inner = -(s0 * s0) + (sqall - s0 * s0)
    denom = jnp.sqrt(jnp.maximum(jnp.abs(-inner), 1e-8)) / jnp.sqrt(CCURV)
    return s / denom


def _tc_a_body(nf_ref, w1sp_ref, w1t_ref, b1_ref, sv_ref, out_ref):
    nf = nf_ref[...]
    sq = jnp.sum(nf * nf, axis=1, keepdims=True)
    nrm = jnp.maximum(jnp.sqrt(sq), 1e-8)
    et = jnp.exp(nrm)
    emt = 1.0 / et
    time = 0.5 * (et + emt)
    space = nf * (0.5 * (et - emt) / nrm)
    h = lax.dot_general(w1sp_ref[...], space, (((1,), (1,)), ((), ())),
                        preferred_element_type=jnp.float32, precision=_prec)
    h = h + lax.dot_general(w1t_ref[...], time, (((1,), (1,)), ((), ())),
                            preferred_element_type=jnp.float32, precision=_prec)
    h = h + b1_ref[...]
    out_ref[...] = _lorentz_normalize(h, sv_ref[0, 0])


def _tc_b_body(sup_ref, w2_ref, b2_ref, sv_ref, out_ref):
    hagg = _agg_normalize(sup_ref[...])
    hr = jnp.maximum(hagg, 0.0)
    h = lax.dot_general(w2_ref[...], hr, (((1,), (0,)), ((), ())),
                        preferred_element_type=jnp.float32, precision=_prec)
    h = h + b2_ref[...]
    out_ref[...] = _lorentz_normalize(h, sv_ref[0, 0])


def _tc_c_body(sup_ref, cls_ref, bias_ref, out_ref):
    h = _agg_normalize(sup_ref[...])
    logits = lax.dot_general(h, cls_ref[...], (((0,), (1,)), ((), ())),
                             preferred_element_type=jnp.float32,
                             precision=_prec)
    out_ref[...] = 2.0 * CCURV + 2.0 * logits + bias_ref[...]


def _tc_stage_a(nf, w1sp, w1t, b1p, sv1):
    return pl.pallas_call(
        _tc_a_body,
        grid=(NP // NB,),
        in_specs=[
            pl.BlockSpec((NB, D), lambda i: (i, 0)),
            pl.BlockSpec((DHP, D), lambda i: (0, 0)),
            pl.BlockSpec((DHP, 1), lambda i: (0, 0)),
            pl.BlockSpec((DHP, 1), lambda i: (0, 0)),
            pl.BlockSpec((1, 1), lambda i: (0, 0)),
        ],
        out_specs=pl.BlockSpec((DHP, NB), lambda i: (0, i)),
        out_shape=jax.ShapeDtypeStruct((DHP, NP), jnp.float32),
    )(nf, w1sp, w1t, b1p, sv1)


def _tc_stage_b(sup, w2p, b2p, sv2):
    return pl.pallas_call(
        _tc_b_body,
        grid=(NP // NB,),
        in_specs=[
            pl.BlockSpec((DHP, NB), lambda i: (0, i)),
            pl.BlockSpec((DHP, DHP), lambda i: (0, 0)),
            pl.BlockSpec((DHP, 1), lambda i: (0, 0)),
            pl.BlockSpec((1, 1), lambda i: (0, 0)),
        ],
        out_specs=pl.BlockSpec((DHP, NB), lambda i: (0, i)),
        out_shape=jax.ShapeDtypeStruct((DHP, NP), jnp.float32),
    )(sup, w2p, b2p, sv2)


def _tc_stage_c(sup, clsm, biasp):
    return pl.pallas_call(
        _tc_c_body,
        grid=(NP // NB,),
        in_specs=[
            pl.BlockSpec((DHP, NB), lambda i: (0, i)),
            pl.BlockSpec((NCLS, DHP), lambda i: (0, 0)),
            pl.BlockSpec((1, NCLS), lambda i: (0, 0)),
        ],
        out_specs=pl.BlockSpec((NB, NCLS), lambda i: (i, 0)),
        out_shape=jax.ShapeDtypeStruct((NP, NCLS), jnp.float32),
    )(sup, clsm, biasp)


def _sc_seg_sum(hT, ei2):
    mesh = plsc.VectorSubcoreMesh(core_axis_name="c", subcore_axis_name="s")

    @functools.partial(
        pl.kernel,
        out_type=jax.ShapeDtypeStruct((DHP, NP), jnp.float32),
        mesh=mesh,
        compiler_params=pltpu.CompilerParams(needs_layout_passes=False),
        scratch_types=[
            pltpu.VMEM((CROWS * NP,), jnp.float32),
            pltpu.VMEM((CROWS * NP,), jnp.float32),
            pltpu.VMEM((2 * ECH,), jnp.int32),
            pltpu.VMEM((2 * ECH,), jnp.int32),
            pltpu.SemaphoreType.DMA,
            pltpu.SemaphoreType.DMA,
        ],
    )
    def seg(hT_hbm, ei2_hbm, out_hbm, tab, acc, bb0, bb1, semA, semB):
        wid = lax.axis_index("s") * NC + lax.axis_index("c")
        r0 = wid * CROWS

        for c in range(CROWS):
            pltpu.sync_copy(hT_hbm.at[r0 + c], tab.at[pl.ds(c * NP, NP)])

        zeros16 = jnp.zeros((16,), jnp.float32)

        def zero_body(j, _):
            for u in range(8):
                acc[pl.ds((j * 8 + u) * 16, 16)] = zeros16
            return 0

        lax.fori_loop(0, CROWS * NP // 128, zero_body, 0)

        def fire(k, buf, sem):
            pltpu.async_copy(ei2_hbm.at[k], buf, sem)

        def wait(buf, sem):
            pltpu.make_async_copy(ei2_hbm.at[0], buf, sem).wait()

        def process(buf):
            def edge_body(j, _):
                for u in range(UNROLL):
                    base = (j * UNROLL + u) * 16
                    s16 = buf[pl.ds(base, 16)]
                    d16 = buf[pl.ds(ECH + base, 16)]
                    for c in range(CROWS):
                        v = plsc.load_gather(tab, [s16 + (c * NP)])
                        plsc.addupdate_scatter(acc, [d16 + (c * NP)], v)
                return 0

            lax.fori_loop(0, ECH // (16 * UNROLL), edge_body, 0)

        fire(0, bb0, semA)

        def pair_body(p, _):
            fire(2 * p + 1, bb1, semB)
            wait(bb0, semA)
            process(bb0)

            @pl.when(p < NCH // 2 - 1)
            def _():
                fire(2 * p + 2, bb0, semA)

            wait(bb1, semB)
            process(bb1)
            return 0

        lax.fori_loop(0, NCH // 2, pair_body, 0)

        for c in range(CROWS):
            pltpu.sync_copy(acc.at[pl.ds(c * NP, NP)], out_hbm.at[r0 + c])

    return seg(hT, ei2)


def kernel(node_feat, edge_index, W1, b1, s1, W2, b2, s2, cls, bias_dec):
    f32 = jnp.float32
    ei2 = jnp.concatenate(
        [edge_index[0].reshape(NCH, ECH), edge_index[1].reshape(NCH, ECH)],
        axis=1)

    w1p = jnp.zeros((DHP, DH), f32).at[:DH].set(W1)
    w1sp = w1p[:, 1:]
    w1t = w1p[:, 0:1]
    b1p = jnp.zeros((DHP, 1), f32).at[:DH, 0].set(b1)
    sv1 = jnp.minimum(jnp.exp(s1), 10.0).reshape(1, 1).astype(f32)
    w2p = jnp.zeros((DHP, DHP), f32).at[:DH, :DH].set(W2)
    b2p = jnp.zeros((DHP, 1), f32).at[:DH, 0].set(b2)
    sv2 = jnp.minimum(jnp.exp(s2), 10.0).reshape(1, 1).astype(f32)
    clsm = jnp.zeros((NCLS, DHP), f32).at[:, :DH].set(cls)
    clsm = clsm.at[:, 0].mul(-1.0)
    biasp = bias_dec.reshape(1, NCLS).astype(f32)

    nfp = jnp.zeros((NP, D), f32).at[:N].set(node_feat)

    h1 = _tc_stage_a(nfp, w1sp, w1t, b1p, sv1)
    sup1 = _sc_seg_sum(h1, ei2)
    h2 = _tc_stage_b(sup1, w2p, b2p, sv2)
    sup2 = _sc_seg_sum(h2, ei2)
    return _tc_stage_c(sup2, clsm, biasp)[:N]


# speedup vs baseline: 4.6660x; 1.6124x over previous
import functools

import jax
import jax.numpy as jnp
from jax import lax
from jax.experimental import pallas as pl
from jax.experimental.pallas import tpu as pltpu
from jax.experimental.pallas import tpu_sc as plsc

N = 10000
NP = 10240
E = 320000
D = 128
DH = D + 1
NCLS = 7
CCURV = 1.0

NC = 2
NS = 16
NW = NC * NS
CROWS = 5
DHP = NW * CROWS
ECH = 4000
NCH = E // ECH
UNROLL = 5
NB = 2048

_prec = jax.lax.Precision.HIGHEST


def _lorentz_normalize(h, sval):
    h0 = h[0:1, :]
    sig = 1.0 / (1.0 + jnp.exp(-h0))
    time = sig * sval + (jnp.sqrt(CCURV) + 0.5)
    sqall = jnp.sum(h * h, axis=0, keepdims=True)
    sq = jnp.maximum(sqall - h0 * h0, 1e-8)
    scale = (time * time - CCURV) / sq
    fac = jnp.sqrt(jnp.maximum(scale, 1e-8))
    rows = lax.broadcasted_iota(jnp.int32, h.shape, 0)
    return jnp.where(rows == 0, time, h * fac)


def _agg_normalize(s):
    s0 = s[0:1, :]
    sqall = jnp.sum(s * s, axis=0, keepdims=True)
    inner = -(s0 * s0) + (sqall - s0 * s0)
    denom = jnp.sqrt(jnp.maximum(jnp.abs(-inner), 1e-8)) / jnp.sqrt(CCURV)
    return s / denom


def _tc_a_body(nf_ref, w1sp_ref, w1t_ref, b1_ref, sv_ref, out_ref):
    nf = nf_ref[...]
    sq = jnp.sum(nf * nf, axis=1, keepdims=True)
    nrm = jnp.maximum(jnp.sqrt(sq), 1e-8)
    et = jnp.exp(nrm)
    emt = 1.0 / et
    time = 0.5 * (et + emt)
    space = nf * (0.5 * (et - emt) / nrm)
    h = lax.dot_general(w1sp_ref[...], space, (((1,), (1,)), ((), ())),
                        preferred_element_type=jnp.float32, precision=_prec)
    h = h + lax.dot_general(w1t_ref[...], time, (((1,), (1,)), ((), ())),
                            preferred_element_type=jnp.float32, precision=_prec)
    h = h + b1_ref[...]
    out_ref[...] = _lorentz_normalize(h, sv_ref[0, 0])


def _tc_b_body(sup_ref, w2_ref, b2_ref, sv_ref, out_ref):
    hagg = _agg_normalize(sup_ref[...])
    hr = jnp.maximum(hagg, 0.0)
    h = lax.dot_general(w2_ref[...], hr, (((1,), (0,)), ((), ())),
                        preferred_element_type=jnp.float32, precision=_prec)
    h = h + b2_ref[...]
    out_ref[...] = _lorentz_normalize(h, sv_ref[0, 0])


def _tc_c_body(sup_ref, cls_ref, bias_ref, out_ref):
    h = _agg_normalize(sup_ref[...])
    logits = lax.dot_general(h, cls_ref[...], (((0,), (1,)), ((), ())),
                             preferred_element_type=jnp.float32,
                             precision=_prec)
    out_ref[...] = 2.0 * CCURV + 2.0 * logits + bias_ref[...]


def _tc_stage_a(nf, w1sp, w1t, b1p, sv1):
    return pl.pallas_call(
        _tc_a_body,
        grid=(NP // NB,),
        in_specs=[
            pl.BlockSpec((NB, D), lambda i: (i, 0)),
            pl.BlockSpec((DHP, D), lambda i: (0, 0)),
            pl.BlockSpec((DHP, 1), lambda i: (0, 0)),
            pl.BlockSpec((DHP, 1), lambda i: (0, 0)),
            pl.BlockSpec((1, 1), lambda i: (0, 0)),
        ],
        out_specs=pl.BlockSpec((DHP, NB), lambda i: (0, i)),
        out_shape=jax.ShapeDtypeStruct((DHP, NP), jnp.float32),
    )(nf, w1sp, w1t, b1p, sv1)


def _tc_stage_b(sup, w2p, b2p, sv2):
    return pl.pallas_call(
        _tc_b_body,
        grid=(NP // NB,),
        in_specs=[
            pl.BlockSpec((DHP, NB), lambda i: (0, i)),
            pl.BlockSpec((DHP, DHP), lambda i: (0, 0)),
            pl.BlockSpec((DHP, 1), lambda i: (0, 0)),
            pl.BlockSpec((1, 1), lambda i: (0, 0)),
        ],
        out_specs=pl.BlockSpec((DHP, NB), lambda i: (0, i)),
        out_shape=jax.ShapeDtypeStruct((DHP, NP), jnp.float32),
    )(sup, w2p, b2p, sv2)


def _tc_stage_c(sup, clsm, biasp):
    return pl.pallas_call(
        _tc_c_body,
        grid=(NP // NB,),
        in_specs=[
            pl.BlockSpec((DHP, NB), lambda i: (0, i)),
            pl.BlockSpec((NCLS, DHP), lambda i: (0, 0)),
            pl.BlockSpec((1, NCLS), lambda i: (0, 0)),
        ],
        out_specs=pl.BlockSpec((NB, NCLS), lambda i: (i, 0)),
        out_shape=jax.ShapeDtypeStruct((NP, NCLS), jnp.float32),
    )(sup, clsm, biasp)


def _sc_seg_sum(hT, ei2):
    mesh = plsc.VectorSubcoreMesh(core_axis_name="c", subcore_axis_name="s")

    @functools.partial(
        pl.kernel,
        out_type=jax.ShapeDtypeStruct((DHP, NP), jnp.float32),
        mesh=mesh,
        compiler_params=pltpu.CompilerParams(needs_layout_passes=False),
        scratch_types=[
            pltpu.VMEM((CROWS * NP,), jnp.float32),
            pltpu.VMEM((CROWS * NP,), jnp.float32),
            pltpu.VMEM((2 * ECH,), jnp.int32),
            pltpu.VMEM((2 * ECH,), jnp.int32),
            pltpu.SemaphoreType.DMA,
            pltpu.SemaphoreType.DMA,
        ],
    )
    def seg(hT_hbm, ei2_hbm, out_hbm, tab, acc, bb0, bb1, semA, semB):
        wid = lax.axis_index("s") * NC + lax.axis_index("c")
        r0 = wid * CROWS

        for c in range(CROWS):
            pltpu.sync_copy(hT_hbm.at[r0 + c], tab.at[pl.ds(c * NP, NP)])

        zeros16 = jnp.zeros((16,), jnp.float32)

        def zero_body(j, _):
            for u in range(8):
                acc[pl.ds((j * 8 + u) * 16, 16)] = zeros16
            return 0

        lax.fori_loop(0, CROWS * NP // 128, zero_body, 0)

        def fire(k, buf, sem):
            pltpu.async_copy(ei2_hbm.at[k], buf, sem)

        def wait(buf, sem):
            pltpu.make_async_copy(ei2_hbm.at[0], buf, sem).wait()

        def process(buf):
            def edge_body(j, _):
                for u in range(UNROLL):
                    base = (j * UNROLL + u) * 16
                    s16 = buf[pl.ds(base, 16)]
                    d16 = buf[pl.ds(ECH + base, 16)]
                    vs = [plsc.load_gather(tab, [s16 + (c * NP)])
                          for c in range(CROWS)]
                    for c in range(CROWS):
                        plsc.addupdate_scatter(acc, [d16 + (c * NP)], vs[c])
                return 0

            lax.fori_loop(0, ECH // (16 * UNROLL), edge_body, 0)

        fire(0, bb0, semA)

        def pair_body(p, _):
            fire(2 * p + 1, bb1, semB)
            wait(bb0, semA)
            process(bb0)

            @pl.when(p < NCH // 2 - 1)
            def _():
                fire(2 * p + 2, bb0, semA)

            wait(bb1, semB)
            process(bb1)
            return 0

        lax.fori_loop(0, NCH // 2, pair_body, 0)

        for c in range(CROWS):
            pltpu.sync_copy(acc.at[pl.ds(c * NP, NP)], out_hbm.at[r0 + c])

    return seg(hT, ei2)


def kernel(node_feat, edge_index, W1, b1, s1, W2, b2, s2, cls, bias_dec):
    f32 = jnp.float32
    ei2 = jnp.concatenate(
        [edge_index[0].reshape(NCH, ECH), edge_index[1].reshape(NCH, ECH)],
        axis=1)

    w1p = jnp.zeros((DHP, DH), f32).at[:DH].set(W1)
    w1sp = w1p[:, 1:]
    w1t = w1p[:, 0:1]
    b1p = jnp.zeros((DHP, 1), f32).at[:DH, 0].set(b1)
    sv1 = jnp.minimum(jnp.exp(s1), 10.0).reshape(1, 1).astype(f32)
    w2p = jnp.zeros((DHP, DHP), f32).at[:DH, :DH].set(W2)
    b2p = jnp.zeros((DHP, 1), f32).at[:DH, 0].set(b2)
    sv2 = jnp.minimum(jnp.exp(s2), 10.0).reshape(1, 1).astype(f32)
    clsm = jnp.zeros((NCLS, DHP), f32).at[:, :DH].set(cls)
    clsm = clsm.at[:, 0].mul(-1.0)
    biasp = bias_dec.reshape(1, NCLS).astype(f32)

    nfp = jnp.zeros((NP, D), f32).at[:N].set(node_feat)

    h1 = _tc_stage_a(nfp, w1sp, w1t, b1p, sv1)
    sup1 = _sc_seg_sum(h1, ei2)
    h2 = _tc_stage_b(sup1, w2p, b2p, sv2)
    sup2 = _sc_seg_sum(h2, ei2)
    return _tc_stage_c(sup2, clsm, biasp)[:N]


# speedup vs baseline: 6.9592x; 1.4915x over previous
import functools

import jax
import jax.numpy as jnp
from jax import lax
from jax.experimental import pallas as pl
from jax.experimental.pallas import tpu as pltpu
from jax.experimental.pallas import tpu_sc as plsc

N = 10000
NP = 10240
E = 320000
D = 128
DH = D + 1
NCLS = 7
CCURV = 1.0

NC = 2
NS = 16
NW = NC * NS
CROWS = 4
SROW = NW * CROWS
FP = 136
ECH = 5040
NCH = 64
EPAD = NCH * ECH - E
UNROLL = 5
NB = 2048

_prec = jax.lax.Precision.HIGHEST


def _lorentz_normalize(h, sval):
    h0 = h[0:1, :]
    sig = 1.0 / (1.0 + jnp.exp(-h0))
    time = sig * sval + (jnp.sqrt(CCURV) + 0.5)
    sqall = jnp.sum(h * h, axis=0, keepdims=True)
    sq = jnp.maximum(sqall - h0 * h0, 1e-8)
    scale = (time * time - CCURV) / sq
    fac = jnp.sqrt(jnp.maximum(scale, 1e-8))
    rows = lax.broadcasted_iota(jnp.int32, h.shape, 0)
    return jnp.where(rows == 0, time, h * fac)


def _agg_normalize(s):
    s0 = s[0:1, :]
    sqall = jnp.sum(s * s, axis=0, keepdims=True)
    inner = -(s0 * s0) + (sqall - s0 * s0)
    denom = jnp.sqrt(jnp.maximum(jnp.abs(-inner), 1e-8)) / jnp.sqrt(CCURV)
    return s / denom


def _assemble(sup_ref, parts_ref):
    s = sup_ref[...]
    p128 = jnp.sum(parts_ref[...], axis=0, keepdims=True)
    rows = lax.broadcasted_iota(jnp.int32, s.shape, 0)
    return jnp.where(rows < SROW, s, jnp.where(rows == SROW, p128, 0.0))


def _tc_a_body(nf_ref, w1sp_ref, w1t_ref, b1_ref, sv_ref, out_ref):
    nf = nf_ref[...]
    sq = jnp.sum(nf * nf, axis=1, keepdims=True)
    nrm = jnp.maximum(jnp.sqrt(sq), 1e-8)
    et = jnp.exp(nrm)
    emt = 1.0 / et
    time = 0.5 * (et + emt)
    space = nf * (0.5 * (et - emt) / nrm)
    h = lax.dot_general(w1sp_ref[...], space, (((1,), (1,)), ((), ())),
                        preferred_element_type=jnp.float32, precision=_prec)
    h = h + lax.dot_general(w1t_ref[...], time, (((1,), (1,)), ((), ())),
                            preferred_element_type=jnp.float32, precision=_prec)
    h = h + b1_ref[...]
    out_ref[...] = _lorentz_normalize(h, sv_ref[0, 0])


def _tc_b_body(sup_ref, parts_ref, w2_ref, b2_ref, sv_ref, out_ref):
    hagg = _agg_normalize(_assemble(sup_ref, parts_ref))
    hr = jnp.maximum(hagg, 0.0)
    h = lax.dot_general(w2_ref[...], hr, (((1,), (0,)), ((), ())),
                        preferred_element_type=jnp.float32, precision=_prec)
    h = h + b2_ref[...]
    out_ref[...] = _lorentz_normalize(h, sv_ref[0, 0])


def _tc_c_body(sup_ref, parts_ref, cls_ref, bias_ref, out_ref):
    h = _agg_normalize(_assemble(sup_ref, parts_ref))
    logits = lax.dot_general(h, cls_ref[...], (((0,), (1,)), ((), ())),
                             preferred_element_type=jnp.float32,
                             precision=_prec)
    out_ref[...] = 2.0 * CCURV + 2.0 * logits + bias_ref[...]


def _tc_stage_a(nf, w1sp, w1t, b1p, sv1):
    return pl.pallas_call(
        _tc_a_body,
        grid=(NP // NB,),
        in_specs=[
            pl.BlockSpec((NB, D), lambda i: (i, 0)),
            pl.BlockSpec((FP, D), lambda i: (0, 0)),
            pl.BlockSpec((FP, 1), lambda i: (0, 0)),
            pl.BlockSpec((FP, 1), lambda i: (0, 0)),
            pl.BlockSpec((1, 1), lambda i: (0, 0)),
        ],
        out_specs=pl.BlockSpec((FP, NB), lambda i: (0, i)),
        out_shape=jax.ShapeDtypeStruct((FP, NP), jnp.float32),
    )(nf, w1sp, w1t, b1p, sv1)


def _tc_stage_b(sup, parts, w2p, b2p, sv2):
    return pl.pallas_call(
        _tc_b_body,
        grid=(NP // NB,),
        in_specs=[
            pl.BlockSpec((FP, NB), lambda i: (0, i)),
            pl.BlockSpec((NW, NB), lambda i: (0, i)),
            pl.BlockSpec((FP, FP), lambda i: (0, 0)),
            pl.BlockSpec((FP, 1), lambda i: (0, 0)),
            pl.BlockSpec((1, 1), lambda i: (0, 0)),
        ],
        out_specs=pl.BlockSpec((FP, NB), lambda i: (0, i)),
        out_shape=jax.ShapeDtypeStruct((FP, NP), jnp.float32),
    )(sup, parts, w2p, b2p, sv2)


def _tc_stage_c(sup, parts, clsm, biasp):
    return pl.pallas_call(
        _tc_c_body,
        grid=(NP // NB,),
        in_specs=[
            pl.BlockSpec((FP, NB), lambda i: (0, i)),
            pl.BlockSpec((NW, NB), lambda i: (0, i)),
            pl.BlockSpec((NCLS, FP), lambda i: (0, 0)),
            pl.BlockSpec((1, NCLS), lambda i: (0, 0)),
        ],
        out_specs=pl.BlockSpec((NB, NCLS), lambda i: (i, 0)),
        out_shape=jax.ShapeDtypeStruct((NP, NCLS), jnp.float32),
    )(sup, parts, clsm, biasp)


def _sc_seg_sum(hT, ei2):
    mesh = plsc.VectorSubcoreMesh(core_axis_name="c", subcore_axis_name="s")

    @functools.partial(
        pl.kernel,
        out_type=(jax.ShapeDtypeStruct((FP, NP), jnp.float32),
                  jax.ShapeDtypeStruct((NW, NP), jnp.float32)),
        mesh=mesh,
        compiler_params=pltpu.CompilerParams(needs_layout_passes=False),
        scratch_types=[
            pltpu.VMEM(((CROWS + 1) * NP,), jnp.float32),
            pltpu.VMEM(((CROWS + 1) * NP,), jnp.float32),
            pltpu.VMEM((ECH,), jnp.int32),
            pltpu.VMEM((ECH,), jnp.int32),
            pltpu.SemaphoreType.DMA,
            pltpu.SemaphoreType.DMA,
        ],
    )
    def seg(hT_hbm, ei2_hbm, out_hbm, parts_hbm, tab, acc, bb0, bb1,
            semA, semB):
        wid = lax.axis_index("s") * NC + lax.axis_index("c")
        r0 = wid * CROWS

        for c in range(CROWS):
            pltpu.sync_copy(hT_hbm.at[r0 + c], tab.at[pl.ds(c * NP, NP)])
        pltpu.sync_copy(hT_hbm.at[SROW], tab.at[pl.ds(CROWS * NP, NP)])

        zeros16 = jnp.zeros((16,), jnp.float32)

        def zero_body(j, _):
            for u in range(8):
                acc[pl.ds((j * 8 + u) * 16, 16)] = zeros16
            return 0

        lax.fori_loop(0, (CROWS + 1) * NP // 128, zero_body, 0)

        tabs = [tab.at[pl.ds(c * NP, NP)] for c in range(CROWS + 1)]
        accs = [acc.at[pl.ds(c * NP, NP)] for c in range(CROWS + 1)]

        def fire(k, buf, sem):
            pltpu.async_copy(ei2_hbm.at[k], buf, sem)

        def wait(buf, sem):
            pltpu.make_async_copy(ei2_hbm.at[0], buf, sem).wait()

        def process(k, buf):
            def edge_body(j, _):
                ss = []
                dd = []
                for u in range(UNROLL):
                    base = (j * UNROLL + u) * 16
                    p16 = buf[pl.ds(base, 16)]
                    ss.append(lax.shift_right_logical(p16, 14))
                    dd.append(lax.bitwise_and(p16, 16383))
                prev = None
                for u in range(UNROLL):
                    vs = [plsc.load_gather(tabs[c], [ss[u]])
                          for c in range(CROWS)]
                    if prev is not None:
                        for c in range(CROWS):
                            plsc.addupdate_scatter(accs[c], [dd[u - 1]],
                                                   prev[c])
                    prev = vs
                for c in range(CROWS):
                    plsc.addupdate_scatter(accs[c], [dd[UNROLL - 1]], prev[c])
                return 0

            lax.fori_loop(0, ECH // (16 * UNROLL), edge_body, 0)

            @pl.when(lax.rem(k, NW) == wid)
            def _():
                def srow_body(j, _):
                    prev = None
                    pd = None
                    for u in range(UNROLL):
                        base = (j * UNROLL + u) * 16
                        p16 = buf[pl.ds(base, 16)]
                        s16 = lax.shift_right_logical(p16, 14)
                        d16 = lax.bitwise_and(p16, 16383)
                        v = plsc.load_gather(tabs[CROWS], [s16])
                        if prev is not None:
                            plsc.addupdate_scatter(accs[CROWS], [pd], prev)
                        prev, pd = v, d16
                    plsc.addupdate_scatter(accs[CROWS], [pd], prev)
                    return 0

                lax.fori_loop(0, ECH // (16 * UNROLL), srow_body, 0)

        fire(0, bb0, semA)

        def pair_body(p, _):
            fire(2 * p + 1, bb1, semB)
            wait(bb0, semA)
            process(2 * p, bb0)

            @pl.when(p < NCH // 2 - 1)
            def _():
                fire(2 * p + 2, bb0, semA)

            wait(bb1, semB)
            process(2 * p + 1, bb1)
            return 0

        lax.fori_loop(0, NCH // 2, pair_body, 0)

        for c in range(CROWS):
            pltpu.sync_copy(acc.at[pl.ds(c * NP, NP)], out_hbm.at[r0 + c])
        pltpu.sync_copy(acc.at[pl.ds(CROWS * NP, NP)], parts_hbm.at[wid])

    return seg(hT, ei2)


def kernel(node_feat, edge_index, W1, b1, s1, W2, b2, s2, cls, bias_dec):
    f32 = jnp.float32

    packed = edge_index[0] * 16384 + edge_index[1]
    pad = jnp.full((EPAD,), NP - 1, jnp.int32)
    ei2 = jnp.concatenate([packed, pad]).reshape(NCH, ECH)

    w1p = jnp.zeros((FP, DH), f32).at[:DH].set(W1)
    w1sp = w1p[:, 1:]
    w1t = w1p[:, 0:1]
    b1p = jnp.zeros((FP, 1), f32).at[:DH, 0].set(b1)
    sv1 = jnp.minimum(jnp.exp(s1), 10.0).reshape(1, 1).astype(f32)
    w2p = jnp.zeros((FP, FP), f32).at[:DH, :DH].set(W2)
    b2p = jnp.zeros((FP, 1), f32).at[:DH, 0].set(b2)
    sv2 = jnp.minimum(jnp.exp(s2), 10.0).reshape(1, 1).astype(f32)
    clsm = jnp.zeros((NCLS, FP), f32).at[:, :DH].set(cls)
    clsm = clsm.at[:, 0].mul(-1.0)
    biasp = bias_dec.reshape(1, NCLS).astype(f32)

    nfp = jnp.zeros((NP, D), f32).at[:N].set(node_feat)

    h1 = _tc_stage_a(nfp, w1sp, w1t, b1p, sv1)
    sup1, parts1 = _sc_seg_sum(h1, ei2)
    h2 = _tc_stage_b(sup1, parts1, w2p, b2p, sv2)
    sup2, parts2 = _sc_seg_sum(h2, ei2)
    return _tc_stage_c(sup2, parts2, clsm, biasp)[:N]


# speedup vs baseline: 7.1396x; 1.0259x over previous
import functools

import jax
import jax.numpy as jnp
from jax import lax
from jax.experimental import pallas as pl
from jax.experimental.pallas import tpu as pltpu
from jax.experimental.pallas import tpu_sc as plsc

N = 10000
NP = 10240
E = 320000
D = 128
DH = D + 1
NCLS = 7
CCURV = 1.0

NC = 2
NS = 16
NW = NC * NS
CROWS = 4
SROW = NW * CROWS
FP = 136
ECH = 10080
NCH = 32
EPAD = NCH * ECH - E
UNROLL = 7
NB = 2048

_prec = jax.lax.Precision.HIGHEST


def _lorentz_normalize(h, sval):
    h0 = h[0:1, :]
    sig = 1.0 / (1.0 + jnp.exp(-h0))
    time = sig * sval + (jnp.sqrt(CCURV) + 0.5)
    sqall = jnp.sum(h * h, axis=0, keepdims=True)
    sq = jnp.maximum(sqall - h0 * h0, 1e-8)
    scale = (time * time - CCURV) / sq
    fac = jnp.sqrt(jnp.maximum(scale, 1e-8))
    rows = lax.broadcasted_iota(jnp.int32, h.shape, 0)
    return jnp.where(rows == 0, time, h * fac)


def _agg_normalize(s):
    s0 = s[0:1, :]
    sqall = jnp.sum(s * s, axis=0, keepdims=True)
    inner = -(s0 * s0) + (sqall - s0 * s0)
    denom = jnp.sqrt(jnp.maximum(jnp.abs(-inner), 1e-8)) / jnp.sqrt(CCURV)
    return s / denom


def _assemble(sup_ref, parts_ref):
    s = sup_ref[...]
    p128 = jnp.sum(parts_ref[...], axis=0, keepdims=True)
    rows = lax.broadcasted_iota(jnp.int32, s.shape, 0)
    return jnp.where(rows < SROW, s, jnp.where(rows == SROW, p128, 0.0))


def _tc_a_body(nf_ref, w1sp_ref, w1t_ref, b1_ref, sv_ref, out_ref):
    nf = nf_ref[...]
    sq = jnp.sum(nf * nf, axis=1, keepdims=True)
    nrm = jnp.maximum(jnp.sqrt(sq), 1e-8)
    et = jnp.exp(nrm)
    emt = 1.0 / et
    time = 0.5 * (et + emt)
    space = nf * (0.5 * (et - emt) / nrm)
    h = lax.dot_general(w1sp_ref[...], space, (((1,), (1,)), ((), ())),
                        preferred_element_type=jnp.float32, precision=_prec)
    h = h + lax.dot_general(w1t_ref[...], time, (((1,), (1,)), ((), ())),
                            preferred_element_type=jnp.float32, precision=_prec)
    h = h + b1_ref[...]
    out_ref[...] = _lorentz_normalize(h, sv_ref[0, 0])


def _tc_b_body(sup_ref, parts_ref, w2_ref, b2_ref, sv_ref, out_ref):
    hagg = _agg_normalize(_assemble(sup_ref, parts_ref))
    hr = jnp.maximum(hagg, 0.0)
    h = lax.dot_general(w2_ref[...], hr, (((1,), (0,)), ((), ())),
                        preferred_element_type=jnp.float32, precision=_prec)
    h = h + b2_ref[...]
    out_ref[...] = _lorentz_normalize(h, sv_ref[0, 0])


def _tc_c_body(sup_ref, parts_ref, cls_ref, bias_ref, out_ref):
    h = _agg_normalize(_assemble(sup_ref, parts_ref))
    logits = lax.dot_general(h, cls_ref[...], (((0,), (1,)), ((), ())),
                             preferred_element_type=jnp.float32,
                             precision=_prec)
    out_ref[...] = 2.0 * CCURV + 2.0 * logits + bias_ref[...]


def _tc_stage_a(nf, w1sp, w1t, b1p, sv1):
    return pl.pallas_call(
        _tc_a_body,
        grid=(NP // NB,),
        in_specs=[
            pl.BlockSpec((NB, D), lambda i: (i, 0)),
            pl.BlockSpec((FP, D), lambda i: (0, 0)),
            pl.BlockSpec((FP, 1), lambda i: (0, 0)),
            pl.BlockSpec((FP, 1), lambda i: (0, 0)),
            pl.BlockSpec((1, 1), lambda i: (0, 0)),
        ],
        out_specs=pl.BlockSpec((FP, NB), lambda i: (0, i)),
        out_shape=jax.ShapeDtypeStruct((FP, NP), jnp.float32),
    )(nf, w1sp, w1t, b1p, sv1)


def _tc_stage_b(sup, parts, w2p, b2p, sv2):
    return pl.pallas_call(
        _tc_b_body,
        grid=(NP // NB,),
        in_specs=[
            pl.BlockSpec((FP, NB), lambda i: (0, i)),
            pl.BlockSpec((NW, NB), lambda i: (0, i)),
            pl.BlockSpec((FP, FP), lambda i: (0, 0)),
            pl.BlockSpec((FP, 1), lambda i: (0, 0)),
            pl.BlockSpec((1, 1), lambda i: (0, 0)),
        ],
        out_specs=pl.BlockSpec((FP, NB), lambda i: (0, i)),
        out_shape=jax.ShapeDtypeStruct((FP, NP), jnp.float32),
    )(sup, parts, w2p, b2p, sv2)


def _tc_stage_c(sup, parts, clsm, biasp):
    return pl.pallas_call(
        _tc_c_body,
        grid=(NP // NB,),
        in_specs=[
            pl.BlockSpec((FP, NB), lambda i: (0, i)),
            pl.BlockSpec((NW, NB), lambda i: (0, i)),
            pl.BlockSpec((NCLS, FP), lambda i: (0, 0)),
            pl.BlockSpec((1, NCLS), lambda i: (0, 0)),
        ],
        out_specs=pl.BlockSpec((NB, NCLS), lambda i: (i, 0)),
        out_shape=jax.ShapeDtypeStruct((NP, NCLS), jnp.float32),
    )(sup, parts, clsm, biasp)


def _sc_seg_sum(hT, ei2):
    mesh = plsc.VectorSubcoreMesh(core_axis_name="c", subcore_axis_name="s")

    @functools.partial(
        pl.kernel,
        out_type=(jax.ShapeDtypeStruct((FP, NP), jnp.float32),
                  jax.ShapeDtypeStruct((NW, NP), jnp.float32)),
        mesh=mesh,
        compiler_params=pltpu.CompilerParams(needs_layout_passes=False),
        scratch_types=[
            pltpu.VMEM(((CROWS + 1) * NP,), jnp.float32),
            pltpu.VMEM(((CROWS + 1) * NP,), jnp.float32),
            pltpu.VMEM((ECH,), jnp.int32),
            pltpu.VMEM((ECH,), jnp.int32),
            pltpu.SemaphoreType.DMA,
            pltpu.SemaphoreType.DMA,
        ],
    )
    def seg(hT_hbm, ei2_hbm, out_hbm, parts_hbm, tab, acc, bb0, bb1,
            semA, semB):
        wid = lax.axis_index("s") * NC + lax.axis_index("c")
        r0 = wid * CROWS

        for c in range(CROWS):
            pltpu.sync_copy(hT_hbm.at[r0 + c], tab.at[pl.ds(c * NP, NP)])
        pltpu.sync_copy(hT_hbm.at[SROW], tab.at[pl.ds(CROWS * NP, NP)])

        zeros16 = jnp.zeros((16,), jnp.float32)

        def zero_body(j, _):
            for u in range(8):
                acc[pl.ds((j * 8 + u) * 16, 16)] = zeros16
            return 0

        lax.fori_loop(0, (CROWS + 1) * NP // 128, zero_body, 0)

        tabs = [tab.at[pl.ds(c * NP, NP)] for c in range(CROWS + 1)]
        accs = [acc.at[pl.ds(c * NP, NP)] for c in range(CROWS + 1)]

        def fire(k, buf, sem):
            pltpu.async_copy(ei2_hbm.at[k], buf, sem)

        def wait(buf, sem):
            pltpu.make_async_copy(ei2_hbm.at[0], buf, sem).wait()

        def process(k, buf):
            def edge_body(j, _):
                ss = []
                dd = []
                for u in range(UNROLL):
                    base = (j * UNROLL + u) * 16
                    p16 = buf[pl.ds(base, 16)]
                    ss.append(lax.shift_right_logical(p16, 14))
                    dd.append(lax.bitwise_and(p16, 16383))
                prev = None
                for u in range(UNROLL):
                    vs = [plsc.load_gather(tabs[c], [ss[u]])
                          for c in range(CROWS)]
                    if prev is not None:
                        for c in range(CROWS):
                            plsc.addupdate_scatter(accs[c], [dd[u - 1]],
                                                   prev[c])
                    prev = vs
                for c in range(CROWS):
                    plsc.addupdate_scatter(accs[c], [dd[UNROLL - 1]], prev[c])
                return 0

            lax.fori_loop(0, ECH // (16 * UNROLL), edge_body, 0)

            @pl.when(lax.rem(k, NW) == wid)
            def _():
                def srow_body(j, _):
                    prev = None
                    pd = None
                    for u in range(UNROLL):
                        base = (j * UNROLL + u) * 16
                        p16 = buf[pl.ds(base, 16)]
                        s16 = lax.shift_right_logical(p16, 14)
                        d16 = lax.bitwise_and(p16, 16383)
                        v = plsc.load_gather(tabs[CROWS], [s16])
                        if prev is not None:
                            plsc.addupdate_scatter(accs[CROWS], [pd], prev)
                        prev, pd = v, d16
                    plsc.addupdate_scatter(accs[CROWS], [pd], prev)
                    return 0

                lax.fori_loop(0, ECH // (16 * UNROLL), srow_body, 0)

        fire(0, bb0, semA)

        def pair_body(p, _):
            fire(2 * p + 1, bb1, semB)
            wait(bb0, semA)
            process(2 * p, bb0)

            @pl.when(p < NCH // 2 - 1)
            def _():
                fire(2 * p + 2, bb0, semA)

            wait(bb1, semB)
            process(2 * p + 1, bb1)
            return 0

        lax.fori_loop(0, NCH // 2, pair_body, 0)

        for c in range(CROWS):
            pltpu.sync_copy(acc.at[pl.ds(c * NP, NP)], out_hbm.at[r0 + c])
        pltpu.sync_copy(acc.at[pl.ds(CROWS * NP, NP)], parts_hbm.at[wid])

    return seg(hT, ei2)


def kernel(node_feat, edge_index, W1, b1, s1, W2, b2, s2, cls, bias_dec):
    f32 = jnp.float32

    packed = edge_index[0] * 16384 + edge_index[1]
    pad = jnp.full((EPAD,), NP - 1, jnp.int32)
    ei2 = jnp.concatenate([packed, pad]).reshape(NCH, ECH)

    w1p = jnp.zeros((FP, DH), f32).at[:DH].set(W1)
    w1sp = w1p[:, 1:]
    w1t = w1p[:, 0:1]
    b1p = jnp.zeros((FP, 1), f32).at[:DH, 0].set(b1)
    sv1 = jnp.minimum(jnp.exp(s1), 10.0).reshape(1, 1).astype(f32)
    w2p = jnp.zeros((FP, FP), f32).at[:DH, :DH].set(W2)
    b2p = jnp.zeros((FP, 1), f32).at[:DH, 0].set(b2)
    sv2 = jnp.minimum(jnp.exp(s2), 10.0).reshape(1, 1).astype(f32)
    clsm = jnp.zeros((NCLS, FP), f32).at[:, :DH].set(cls)
    clsm = clsm.at[:, 0].mul(-1.0)
    biasp = bias_dec.reshape(1, NCLS).astype(f32)

    nfp = jnp.zeros((NP, D), f32).at[:N].set(node_feat)

    h1 = _tc_stage_a(nfp, w1sp, w1t, b1p, sv1)
    sup1, parts1 = _sc_seg_sum(h1, ei2)
    h2 = _tc_stage_b(sup1, parts1, w2p, b2p, sv2)
    sup2, parts2 = _sc_seg_sum(h2, ei2)
    return _tc_stage_c(sup2, parts2, clsm, biasp)[:N]


# speedup vs baseline: 7.3656x; 1.0317x over previous
import functools

import jax
import jax.numpy as jnp
from jax import lax
from jax.experimental import pallas as pl
from jax.experimental.pallas import tpu as pltpu
from jax.experimental.pallas import tpu_sc as plsc

N = 10000
NP = 10240
E = 320000
D = 128
DH = D + 1
NCLS = 7
CCURV = 1.0

NC = 2
NS = 16
NW = NC * NS
CROWS = 4
SROW = NW * CROWS
FP = 136
ECH = 10080
NCH = 32
EPAD = NCH * ECH - E
UNROLL = 9
NB = 2048

_prec = jax.lax.Precision.HIGHEST


def _lorentz_normalize(h, sval):
    h0 = h[0:1, :]
    sig = 1.0 / (1.0 + jnp.exp(-h0))
    time = sig * sval + (jnp.sqrt(CCURV) + 0.5)
    sqall = jnp.sum(h * h, axis=0, keepdims=True)
    sq = jnp.maximum(sqall - h0 * h0, 1e-8)
    scale = (time * time - CCURV) / sq
    fac = jnp.sqrt(jnp.maximum(scale, 1e-8))
    rows = lax.broadcasted_iota(jnp.int32, h.shape, 0)
    return jnp.where(rows == 0, time, h * fac)


def _agg_normalize(s):
    s0 = s[0:1, :]
    sqall = jnp.sum(s * s, axis=0, keepdims=True)
    inner = -(s0 * s0) + (sqall - s0 * s0)
    denom = jnp.sqrt(jnp.maximum(jnp.abs(-inner), 1e-8)) / jnp.sqrt(CCURV)
    return s / denom


def _assemble(sup_ref, parts_ref):
    s = sup_ref[...]
    p128 = jnp.sum(parts_ref[...], axis=0, keepdims=True)
    rows = lax.broadcasted_iota(jnp.int32, s.shape, 0)
    return jnp.where(rows < SROW, s, jnp.where(rows == SROW, p128, 0.0))


def _tc_a_body(nf_ref, w1sp_ref, w1t_ref, b1_ref, sv_ref, out_ref):
    nf = nf_ref[...]
    sq = jnp.sum(nf * nf, axis=1, keepdims=True)
    nrm = jnp.maximum(jnp.sqrt(sq), 1e-8)
    et = jnp.exp(nrm)
    emt = 1.0 / et
    time = 0.5 * (et + emt)
    space = nf * (0.5 * (et - emt) / nrm)
    h = lax.dot_general(w1sp_ref[...], space, (((1,), (1,)), ((), ())),
                        preferred_element_type=jnp.float32, precision=_prec)
    h = h + lax.dot_general(w1t_ref[...], time, (((1,), (1,)), ((), ())),
                            preferred_element_type=jnp.float32, precision=_prec)
    h = h + b1_ref[...]
    out_ref[...] = _lorentz_normalize(h, sv_ref[0, 0])


def _tc_b_body(sup_ref, parts_ref, w2_ref, b2_ref, sv_ref, out_ref):
    hagg = _agg_normalize(_assemble(sup_ref, parts_ref))
    hr = jnp.maximum(hagg, 0.0)
    h = lax.dot_general(w2_ref[...], hr, (((1,), (0,)), ((), ())),
                        preferred_element_type=jnp.float32, precision=_prec)
    h = h + b2_ref[...]
    out_ref[...] = _lorentz_normalize(h, sv_ref[0, 0])


def _tc_c_body(sup_ref, parts_ref, cls_ref, bias_ref, out_ref):
    h = _agg_normalize(_assemble(sup_ref, parts_ref))
    logits = lax.dot_general(h, cls_ref[...], (((0,), (1,)), ((), ())),
                             preferred_element_type=jnp.float32,
                             precision=_prec)
    out_ref[...] = 2.0 * CCURV + 2.0 * logits + bias_ref[...]


def _tc_stage_a(nf, w1sp, w1t, b1p, sv1):
    return pl.pallas_call(
        _tc_a_body,
        grid=(NP // NB,),
        in_specs=[
            pl.BlockSpec((NB, D), lambda i: (i, 0)),
            pl.BlockSpec((FP, D), lambda i: (0, 0)),
            pl.BlockSpec((FP, 1), lambda i: (0, 0)),
            pl.BlockSpec((FP, 1), lambda i: (0, 0)),
            pl.BlockSpec((1, 1), lambda i: (0, 0)),
        ],
        out_specs=pl.BlockSpec((FP, NB), lambda i: (0, i)),
        out_shape=jax.ShapeDtypeStruct((FP, NP), jnp.float32),
    )(nf, w1sp, w1t, b1p, sv1)


def _tc_stage_b(sup, parts, w2p, b2p, sv2):
    return pl.pallas_call(
        _tc_b_body,
        grid=(NP // NB,),
        in_specs=[
            pl.BlockSpec((FP, NB), lambda i: (0, i)),
            pl.BlockSpec((NW, NB), lambda i: (0, i)),
            pl.BlockSpec((FP, FP), lambda i: (0, 0)),
            pl.BlockSpec((FP, 1), lambda i: (0, 0)),
            pl.BlockSpec((1, 1), lambda i: (0, 0)),
        ],
        out_specs=pl.BlockSpec((FP, NB), lambda i: (0, i)),
        out_shape=jax.ShapeDtypeStruct((FP, NP), jnp.float32),
    )(sup, parts, w2p, b2p, sv2)


def _tc_stage_c(sup, parts, clsm, biasp):
    return pl.pallas_call(
        _tc_c_body,
        grid=(NP // NB,),
        in_specs=[
            pl.BlockSpec((FP, NB), lambda i: (0, i)),
            pl.BlockSpec((NW, NB), lambda i: (0, i)),
            pl.BlockSpec((NCLS, FP), lambda i: (0, 0)),
            pl.BlockSpec((1, NCLS), lambda i: (0, 0)),
        ],
        out_specs=pl.BlockSpec((NB, NCLS), lambda i: (i, 0)),
        out_shape=jax.ShapeDtypeStruct((NP, NCLS), jnp.float32),
    )(sup, parts, clsm, biasp)


def _sc_seg_sum(hT, ei2):
    mesh = plsc.VectorSubcoreMesh(core_axis_name="c", subcore_axis_name="s")

    @functools.partial(
        pl.kernel,
        out_type=(jax.ShapeDtypeStruct((FP, NP), jnp.float32),
                  jax.ShapeDtypeStruct((NW, NP), jnp.float32)),
        mesh=mesh,
        compiler_params=pltpu.CompilerParams(needs_layout_passes=False),
        scratch_types=[
            pltpu.VMEM(((CROWS + 1) * NP,), jnp.float32),
            pltpu.VMEM(((CROWS + 1) * NP,), jnp.float32),
            pltpu.VMEM((ECH,), jnp.int32),
            pltpu.VMEM((ECH,), jnp.int32),
            pltpu.SemaphoreType.DMA,
            pltpu.SemaphoreType.DMA,
        ],
    )
    def seg(hT_hbm, ei2_hbm, out_hbm, parts_hbm, tab, acc, bb0, bb1,
            semA, semB):
        wid = lax.axis_index("s") * NC + lax.axis_index("c")
        r0 = wid * CROWS

        pltpu.async_copy(ei2_hbm.at[0], bb0, semA)
        for c in range(CROWS):
            pltpu.async_copy(hT_hbm.at[r0 + c], tab.at[pl.ds(c * NP, NP)],
                             semB)
        pltpu.async_copy(hT_hbm.at[SROW], tab.at[pl.ds(CROWS * NP, NP)],
                         semB)

        zeros16 = jnp.zeros((16,), jnp.float32)

        def zero_body(j, _):
            for u in range(8):
                acc[pl.ds((j * 8 + u) * 16, 16)] = zeros16
            return 0

        lax.fori_loop(0, (CROWS + 1) * NP // 128, zero_body, 0)

        for c in range(CROWS + 1):
            pltpu.make_async_copy(hT_hbm.at[SROW],
                                  tab.at[pl.ds(c * NP, NP)], semB).wait()

        tabs = [tab.at[pl.ds(c * NP, NP)] for c in range(CROWS + 1)]
        accs = [acc.at[pl.ds(c * NP, NP)] for c in range(CROWS + 1)]

        def fire(k, buf, sem):
            pltpu.async_copy(ei2_hbm.at[k], buf, sem)

        def wait(buf, sem):
            pltpu.make_async_copy(ei2_hbm.at[0], buf, sem).wait()

        def process(k, buf):
            def edge_body(j, _):
                ss = []
                dd = []
                for u in range(UNROLL):
                    base = (j * UNROLL + u) * 16
                    p16 = buf[pl.ds(base, 16)]
                    ss.append(lax.shift_right_logical(p16, 14))
                    dd.append(lax.bitwise_and(p16, 16383))
                prev = None
                for u in range(UNROLL):
                    vs = [plsc.load_gather(tabs[c], [ss[u]])
                          for c in range(CROWS)]
                    if prev is not None:
                        for c in range(CROWS):
                            plsc.addupdate_scatter(accs[c], [dd[u - 1]],
                                                   prev[c])
                    prev = vs
                for c in range(CROWS):
                    plsc.addupdate_scatter(accs[c], [dd[UNROLL - 1]], prev[c])
                return 0

            lax.fori_loop(0, ECH // (16 * UNROLL), edge_body, 0)

            @pl.when(lax.rem(k, NW) == wid)
            def _():
                def srow_body(j, _):
                    prev = None
                    pd = None
                    for u in range(UNROLL):
                        base = (j * UNROLL + u) * 16
                        p16 = buf[pl.ds(base, 16)]
                        s16 = lax.shift_right_logical(p16, 14)
                        d16 = lax.bitwise_and(p16, 16383)
                        v = plsc.load_gather(tabs[CROWS], [s16])
                        if prev is not None:
                            plsc.addupdate_scatter(accs[CROWS], [pd], prev)
                        prev, pd = v, d16
                    plsc.addupdate_scatter(accs[CROWS], [pd], prev)
                    return 0

                lax.fori_loop(0, ECH // (16 * UNROLL), srow_body, 0)

        def pair_body(p, _):
            fire(2 * p + 1, bb1, semB)
            wait(bb0, semA)
            process(2 * p, bb0)

            @pl.when(p < NCH // 2 - 1)
            def _():
                fire(2 * p + 2, bb0, semA)

            wait(bb1, semB)
            process(2 * p + 1, bb1)
            return 0

        lax.fori_loop(0, NCH // 2, pair_body, 0)

        for c in range(CROWS):
            pltpu.sync_copy(acc.at[pl.ds(c * NP, NP)], out_hbm.at[r0 + c])
        pltpu.sync_copy(acc.at[pl.ds(CROWS * NP, NP)], parts_hbm.at[wid])

    return seg(hT, ei2)


def kernel(node_feat, edge_index, W1, b1, s1, W2, b2, s2, cls, bias_dec):
    f32 = jnp.float32

    packed = edge_index[0] * 16384 + edge_index[1]
    pad = jnp.full((EPAD,), NP - 1, jnp.int32)
    ei2 = jnp.concatenate([packed, pad]).reshape(NCH, ECH)

    w1p = jnp.zeros((FP, DH), f32).at[:DH].set(W1)
    w1sp = w1p[:, 1:]
    w1t = w1p[:, 0:1]
    b1p = jnp.zeros((FP, 1), f32).at[:DH, 0].set(b1)
    sv1 = jnp.minimum(jnp.exp(s1), 10.0).reshape(1, 1).astype(f32)
    w2p = jnp.zeros((FP, FP), f32).at[:DH, :DH].set(W2)
    b2p = jnp.zeros((FP, 1), f32).at[:DH, 0].set(b2)
    sv2 = jnp.minimum(jnp.exp(s2), 10.0).reshape(1, 1).astype(f32)
    clsm = jnp.zeros((NCLS, FP), f32).at[:, :DH].set(cls)
    clsm = clsm.at[:, 0].mul(-1.0)
    biasp = bias_dec.reshape(1, NCLS).astype(f32)

    nfp = jnp.zeros((NP, D), f32).at[:N].set(node_feat)

    h1 = _tc_stage_a(nfp, w1sp, w1t, b1p, sv1)
    sup1, parts1 = _sc_seg_sum(h1, ei2)
    h2 = _tc_stage_b(sup1, parts1, w2p, b2p, sv2)
    sup2, parts2 = _sc_seg_sum(h2, ei2)
    return _tc_stage_c(sup2, parts2, clsm, biasp)[:N]
